# baseline (speedup 1.0000x reference)
import functools

import jax
import jax.numpy as jnp
from jax import lax
from jax.experimental import pallas as pl
from jax.experimental.pallas import tpu as pltpu
from jax.experimental.pallas import tpu_sc as plsc

BATCH = 4096
HIST = 200
EMBED = 128
NUM_EDGE_TYPE = 64
N_ROWS = BATCH * HIST
NUM_WORKERS = 32
ROWS_PER_W = N_ROWS // NUM_WORKERS
CHUNK = 256
N_CHUNKS = ROWS_PER_W // CHUNK
NBUF = 2
SHIP = 1

_mesh = plsc.VectorSubcoreMesh(core_axis_name="c", subcore_axis_name="s")


@functools.partial(
    pl.kernel,
    mesh=_mesh,
    out_type=jax.ShapeDtypeStruct((N_ROWS, EMBED), jnp.float32),
    compiler_params=pltpu.CompilerParams(needs_layout_passes=False),
    scratch_types=(
        [pltpu.VMEM_SHARED((NUM_EDGE_TYPE, EMBED), jnp.float32),
         pltpu.VMEM((NBUF * (CHUNK // 128), 128), jnp.int32),
         pltpu.VMEM((NBUF * CHUNK, EMBED), jnp.float32)]
        + [pltpu.SemaphoreType.DMA] * (3 * NBUF)
    ),
)
def _gather(idx_hbm, table_hbm, out_hbm, table_sh, idx_v, rows_v, *sems):
    gsems, osems, isems = sems[:NBUF], sems[NBUF:2 * NBUF], sems[2 * NBUF:]
    sid = lax.axis_index("s")
    wid = sid * 2 + lax.axis_index("c")
    KI = CHUNK // 128
    irow_base = wid * N_CHUNKS * KI
    out_base = wid * ROWS_PER_W

    pl.when(sid == 0)(lambda: pltpu.sync_copy(table_hbm, table_sh))
    plsc.subcore_barrier()

    def idesc(c, b):
        return pltpu.make_async_copy(
            idx_hbm.at[pl.ds(irow_base + c * KI, KI)],
            idx_v.at[pl.ds(b * KI, KI)],
            isems[b])

    def gdescs(b):
        return [
            pltpu.make_async_copy(
                table_sh.at[idx_v.at[b * KI + j]],
                rows_v.at[pl.ds(b * CHUNK + j * 128, 128)],
                gsems[b])
            for j in range(KI)]

    def gstart(b):
        for d in gdescs(b):
            d.start()

    def gwait(b):
        for d in gdescs(b):
            d.wait()

    def odesc(c, b):
        return pltpu.make_async_copy(
            rows_v.at[pl.ds(b * CHUNK, CHUNK)],
            out_hbm.at[pl.ds(out_base + c * CHUNK, CHUNK)],
            osems[b])

    for b in range(NBUF):
        idesc(b, b).start()

    n_groups = N_CHUNKS // NBUF

    def body(g, carry):
        cb = NBUF * g
        for b in range(NBUF):
            c = cb + b
            pl.when(g > 0)(lambda: odesc(c - NBUF, b).wait())
            idesc(c, b).wait()
            gstart(b)
            b2 = (b - SHIP) % NBUF
            c2 = c - SHIP

            def ship():
                gwait(b2)
                odesc(c2, b2).start()

            def prefetch():
                idesc(c2 + NBUF, b2).start()

            pl.when(c >= SHIP)(ship)
            pl.when((c >= SHIP) & (c2 + NBUF < N_CHUNKS))(prefetch)
        return carry

    lax.fori_loop(0, n_groups, body, 0)

    for c in range(N_CHUNKS - SHIP, N_CHUNKS):
        gwait(c % NBUF)
        odesc(c, c % NBUF).start()
    for c in range(N_CHUNKS - NBUF, N_CHUNKS):
        odesc(c, c % NBUF).wait()


def kernel(data, edge_type_embedding):
    idx = data.reshape(N_ROWS // 128, 128)
    out = _gather(idx, edge_type_embedding)
    return out.reshape(BATCH, HIST, EMBED)


# speedup vs baseline: 1.0193x; 1.0193x over previous
import functools

import jax
import jax.numpy as jnp
from jax import lax
from jax.experimental import pallas as pl
from jax.experimental.pallas import tpu as pltpu
from jax.experimental.pallas import tpu_sc as plsc

BATCH = 4096
HIST = 200
EMBED = 128
NUM_EDGE_TYPE = 64
N_ROWS = BATCH * HIST
NUM_WORKERS = 32
ROWS_PER_W = N_ROWS // NUM_WORKERS
CHUNK = 128
N_CHUNKS = ROWS_PER_W // CHUNK
NBUF = 5
SHIP = 2

_mesh = plsc.VectorSubcoreMesh(core_axis_name="c", subcore_axis_name="s")


@functools.partial(
    pl.kernel,
    mesh=_mesh,
    out_type=jax.ShapeDtypeStruct((N_ROWS, EMBED), jnp.float32),
    compiler_params=pltpu.CompilerParams(needs_layout_passes=False),
    scratch_types=(
        [pltpu.VMEM_SHARED((NUM_EDGE_TYPE, EMBED), jnp.float32),
         pltpu.VMEM((NBUF * (CHUNK // 128), 128), jnp.int32),
         pltpu.VMEM((NBUF * CHUNK, EMBED), jnp.float32)]
        + [pltpu.SemaphoreType.DMA] * (3 * NBUF)
    ),
)
def _gather(idx_hbm, table_hbm, out_hbm, table_sh, idx_v, rows_v, *sems):
    gsems, osems, isems = sems[:NBUF], sems[NBUF:2 * NBUF], sems[2 * NBUF:]
    sid = lax.axis_index("s")
    wid = sid * 2 + lax.axis_index("c")
    KI = CHUNK // 128
    irow_base = wid * N_CHUNKS * KI
    out_base = wid * ROWS_PER_W

    pl.when(sid == 0)(lambda: pltpu.sync_copy(table_hbm, table_sh))
    plsc.subcore_barrier()

    def idesc(c, b):
        return pltpu.make_async_copy(
            idx_hbm.at[pl.ds(irow_base + c * KI, KI)],
            idx_v.at[pl.ds(b * KI, KI)],
            isems[b])

    def gdescs(b):
        return [
            pltpu.make_async_copy(
                table_sh.at[idx_v.at[b * KI + j]],
                rows_v.at[pl.ds(b * CHUNK + j * 128, 128)],
                gsems[b])
            for j in range(KI)]

    def gstart(b):
        for d in gdescs(b):
            d.start()

    def gwait(b):
        for d in gdescs(b):
            d.wait()

    def odesc(c, b):
        return pltpu.make_async_copy(
            rows_v.at[pl.ds(b * CHUNK, CHUNK)],
            out_hbm.at[pl.ds(out_base + c * CHUNK, CHUNK)],
            osems[b])

    for b in range(NBUF):
        idesc(b, b).start()

    n_groups = N_CHUNKS // NBUF

    def body(g, carry):
        cb = NBUF * g
        for b in range(NBUF):
            c = cb + b
            pl.when(g > 0)(lambda: odesc(c - NBUF, b).wait())
            idesc(c, b).wait()
            gstart(b)
            b2 = (b - SHIP) % NBUF
            c2 = c - SHIP

            def ship():
                gwait(b2)
                odesc(c2, b2).start()

            def prefetch():
                idesc(c2 + NBUF, b2).start()

            pl.when(c >= SHIP)(ship)
            pl.when((c >= SHIP) & (c2 + NBUF < N_CHUNKS))(prefetch)
        return carry

    lax.fori_loop(0, n_groups, body, 0)

    for c in range(N_CHUNKS - SHIP, N_CHUNKS):
        gwait(c % NBUF)
        odesc(c, c % NBUF).start()
    for c in range(N_CHUNKS - NBUF, N_CHUNKS):
        odesc(c, c % NBUF).wait()


def kernel(data, edge_type_embedding):
    idx = data.reshape(N_ROWS // 128, 128)
    out = _gather(idx, edge_type_embedding)
    return out.reshape(BATCH, HIST, EMBED)
